# baseline (speedup 1.0000x reference)
import jax
import jax.numpy as jnp
from jax.experimental import pallas as pl
from jax.experimental.pallas import tpu as pltpu

N = 16384
D = 64
BR = 32
G = D // BR


def _tc_body(x_ref, add_ref, mul_ref, mean_ref, rowc_ref, acc_ref):
    g = pl.program_id(0)

    @pl.when(g == 0)
    def _init():
        rowc_ref[...] = jax.lax.broadcasted_iota(
            jnp.int32, (BR, N), 1).astype(jnp.float32) + 2.0
        acc_ref[...] = jnp.zeros((BR, 128), jnp.float32)

    x = x_ref[...]
    add_ref[...] = x + rowc_ref[...]
    mul_ref[...] = x * 3.0
    acc_ref[...] += jnp.sum(x.reshape(BR, N // 128, 128), axis=1)

    @pl.when(g == G - 1)
    def _fin():
        total = jnp.sum(acc_ref[...])
        mean_ref[0, 0] = total / (N * D) + (2.0 + (N - 1) / 2.0)


def _tc_kernel(xt):
    return pl.pallas_call(
        _tc_body,
        grid=(G,),
        in_specs=[pl.BlockSpec((BR, N), lambda g: (g, 0))],
        out_specs=[
            pl.BlockSpec((BR, N), lambda g: (g, 0)),
            pl.BlockSpec((BR, N), lambda g: (g, 0)),
            pl.BlockSpec(memory_space=pltpu.SMEM),
        ],
        out_shape=[
            jax.ShapeDtypeStruct((D, N), jnp.float32),
            jax.ShapeDtypeStruct((D, N), jnp.float32),
            jax.ShapeDtypeStruct((1, 1), jnp.float32),
        ],
        scratch_shapes=[
            pltpu.VMEM((BR, N), jnp.float32),
            pltpu.VMEM((BR, 128), jnp.float32),
        ],
        compiler_params=pltpu.CompilerParams(
            dimension_semantics=("arbitrary",),
        ),
    )(xt)


def kernel(x):
    add_t, mul_t, mean2d = _tc_kernel(x.T)
    return (add_t.T, mul_t.T, mean2d.reshape(()))


# speedup vs baseline: 1.0073x; 1.0073x over previous
import jax
import jax.numpy as jnp
from jax.experimental import pallas as pl
from jax.experimental.pallas import tpu as pltpu

N = 16384
D = 64
BR = 32
G = D // BR


def _tc_body(x_ref, add_ref, mul_ref, mean_ref, rowc_ref, acc_ref):
    g = pl.program_id(0)

    @pl.when(g == 0)
    def _init():
        rowc_ref[...] = jax.lax.broadcasted_iota(
            jnp.int32, (BR, N), 1).astype(jnp.float32) + 2.0
        acc_ref[...] = jnp.zeros((BR, 128), jnp.float32)

    x = x_ref[...]
    add_ref[...] = x + rowc_ref[...]
    mul_ref[...] = x * 3.0
    acc_ref[...] += jnp.sum(x.reshape(BR, N // 128, 128), axis=1)

    @pl.when(g == G - 1)
    def _fin():
        total = jnp.sum(acc_ref[...])
        mean_ref[0, 0] = total / (N * D) + (2.0 + (N - 1) / 2.0)


def _tc_kernel(xt):
    return pl.pallas_call(
        _tc_body,
        grid=(G,),
        in_specs=[pl.BlockSpec((BR, N), lambda g: (g, 0))],
        out_specs=[
            pl.BlockSpec((BR, N), lambda g: (g, 0)),
            pl.BlockSpec((BR, N), lambda g: (g, 0)),
            pl.BlockSpec(memory_space=pltpu.SMEM),
        ],
        out_shape=[
            jax.ShapeDtypeStruct((D, N), jnp.float32),
            jax.ShapeDtypeStruct((D, N), jnp.float32),
            jax.ShapeDtypeStruct((1, 1), jnp.float32),
        ],
        scratch_shapes=[
            pltpu.VMEM((BR, N), jnp.float32),
            pltpu.VMEM((BR, 128), jnp.float32),
        ],
    )(xt)


def kernel(x):
    add_t, mul_t, mean2d = _tc_kernel(x.T)
    return (add_t.T, mul_t.T, mean2d.reshape(()))
